# baseline (speedup 1.0000x reference)
import functools

import jax
import jax.numpy as jnp
from jax import lax
from jax.experimental import pallas as pl
from jax.experimental.pallas import tpu as pltpu
from jax.experimental.pallas import tpu_sc as plsc

N = 8192
NC = 2
NS = 16
NW = NC * NS
NTC = 6144
NSC = N - NTC
XPW = NSC // NW
XB = 4
L = 32
W = 16
NYV = N // L
NP = N // 2

BX = 256
NXB = NTC // BX

_mesh = plsc.VectorSubcoreMesh(core_axis_name="c", subcore_axis_name="s")


@functools.partial(
    pl.kernel,
    mesh=_mesh,
    compiler_params=pltpu.CompilerParams(needs_layout_passes=False),
    out_type=[
        jax.ShapeDtypeStruct((NW, XPW * W), jnp.int32),
        jax.ShapeDtypeStruct((NW, NP), jnp.int32),
    ],
    scratch_types=[
        pltpu.VMEM((3, NP), jnp.int32),
        pltpu.VMEM((XPW * 3,), jnp.int32),
        pltpu.VMEM((NP,), jnp.int32),
        pltpu.VMEM((XPW * W,), jnp.int32),
    ],
)
def _chamfer_sc(x_hbm, y_hbm, cx_out, cy_out, y_v, x_v, cy_v, cxa_v):
    cid = lax.axis_index("c")
    sid = lax.axis_index("s")
    wid = sid * NC + cid

    pltpu.sync_copy(y_hbm, y_v)
    pltpu.sync_copy(x_hbm.at[wid], x_v)

    infv = jnp.full((L,), jnp.inf, jnp.bfloat16)
    infw = plsc.bitcast(infv, jnp.int32)

    def init_body(j, carry):
        cy_v[pl.ds(j * W, W)] = infw
        return carry

    lax.fori_loop(0, NYV, init_body, 0)

    def outer(ib, carry):
        xv = [x_v[pl.ds(ib * 48 + 16 * v, 16)] for v in range(3)]
        for sb in range(16 // XB):
            def bcast(k, c):
                flat = 3 * (sb * XB + k) + c
                return plsc.bitcast(
                    jnp.full((W,), xv[flat // 16][flat % 16], jnp.int32),
                    jnp.bfloat16)
            xs = [[bcast(k, c) for c in range(3)] for k in range(XB)]

            def inner(j, accs):
                y0 = plsc.bitcast(y_v[0, pl.ds(j * W, W)], jnp.bfloat16)
                y1 = plsc.bitcast(y_v[1, pl.ds(j * W, W)], jnp.bfloat16)
                y2 = plsc.bitcast(y_v[2, pl.ds(j * W, W)], jnp.bfloat16)
                ds = []
                naccs = []
                for k in range(XB):
                    d = (jnp.abs(y0 - xs[k][0])
                         + jnp.abs(y1 - xs[k][1])
                         + jnp.abs(y2 - xs[k][2]))
                    ds.append(d)
                    naccs.append(jnp.minimum(accs[k], d))
                m = jnp.minimum(jnp.minimum(ds[0], ds[1]),
                                jnp.minimum(ds[2], ds[3]))
                cyv = plsc.bitcast(cy_v[pl.ds(j * W, W)], jnp.bfloat16)
                cy_v[pl.ds(j * W, W)] = plsc.bitcast(
                    jnp.minimum(cyv, m), jnp.int32)
                return tuple(naccs)

            accs = lax.fori_loop(0, NYV, inner,
                                 tuple(infv for _ in range(XB)))
            for k in range(XB):
                p = ib * 16 + sb * XB + k
                cxa_v[pl.ds(p * W, W)] = plsc.bitcast(accs[k], jnp.int32)
        return carry

    lax.fori_loop(0, XPW // 16, outer, 0)

    pltpu.sync_copy(cxa_v, cx_out.at[wid])
    pltpu.sync_copy(cy_v, cy_out.at[wid])


def _tc_body(p1_ref, p2_ref, y_ref, cxs_ref, cyf_ref, cy_ref):
    j = pl.program_id(0)
    xs = p1_ref[0] + p2_ref[0]
    x0 = xs[:, 0:1].astype(jnp.bfloat16)
    x1 = xs[:, 1:2].astype(jnp.bfloat16)
    x2 = xs[:, 2:3].astype(jnp.bfloat16)
    y0 = y_ref[0, :][None, :]
    y1 = y_ref[1, :][None, :]
    y2 = y_ref[2, :][None, :]
    d = (jnp.abs(x0 - y0) + jnp.abs(x1 - y1) + jnp.abs(x2 - y2))
    bsum = jnp.sum(jnp.min(d, axis=1).astype(jnp.float32))

    @pl.when(j == 0)
    def _():
        cxs_ref[0, 0] = bsum

    @pl.when(j > 0)
    def _():
        cxs_ref[0, 0] = cxs_ref[0, 0] + bsum

    m = d[0:8, :]
    for g in range(1, BX // 8):
        m = jnp.minimum(m, d[8 * g:8 * (g + 1), :])

    mf = m.astype(jnp.float32)

    @pl.when(j == 0)
    def _():
        cy_ref[...] = mf

    @pl.when(j > 0)
    def _():
        cy_ref[...] = jnp.minimum(cy_ref[...], mf)

    @pl.when(j == NXB - 1)
    def _():
        cyf_ref[...] = jnp.min(cy_ref[...], axis=0)


_chamfer_tc = pl.pallas_call(
    _tc_body,
    grid=(NXB,),
    in_specs=[
        pl.BlockSpec((1, BX, 3), lambda j: (0, j, 0)),
        pl.BlockSpec((1, BX, 3), lambda j: (0, j, 0)),
        pl.BlockSpec((3, N), lambda j: (0, 0)),
    ],
    out_specs=[
        pl.BlockSpec((1, 1), lambda j: (0, 0), memory_space=pltpu.SMEM),
        pl.BlockSpec((N,), lambda j: (0,)),
        pl.BlockSpec((8, N), lambda j: (0, 0)),
    ],
    out_shape=[
        jax.ShapeDtypeStruct((1, 1), jnp.float32),
        jax.ShapeDtypeStruct((N,), jnp.float32),
        jax.ShapeDtypeStruct((8, N), jnp.float32),
    ],
)


def _round_bf16_bits(u):
    u16 = jnp.uint32(16)
    r = lax.shift_right_logical(u, u16) & jnp.uint32(1)
    return lax.shift_right_logical(u + jnp.uint32(0x7FFF) + r, u16)


def _lo_f32(w):
    return lax.bitcast_convert_type(w << jnp.int32(16), jnp.float32)


def _hi_f32(w):
    return lax.bitcast_convert_type(w & jnp.int32(-65536), jnp.float32)


def kernel(pc1, est_flow, pc2):
    y = pc2[0]
    ybt = y.astype(jnp.bfloat16).T

    ylo = lax.bitcast_convert_type(ybt[:, :NP], jnp.uint16).astype(jnp.uint32)
    yhi = lax.bitcast_convert_type(ybt[:, NP:], jnp.uint16).astype(jnp.uint32)
    yr = lax.bitcast_convert_type(ylo | (yhi << jnp.uint32(16)),
                                 jnp.int32)

    xsc = (pc1[0, NTC:, :] + est_flow[0, NTC:, :]).reshape(NSC * 3)
    xb = _round_bf16_bits(lax.bitcast_convert_type(xsc, jnp.uint32))
    xr = lax.bitcast_convert_type(
        xb | (xb << jnp.uint32(16)), jnp.int32).reshape(NW, XPW * 3)

    cx_vecs, cy_sc = _chamfer_sc(xr, yr)
    cxs_tc, cyt, _cy8 = _chamfer_tc(pc1, est_flow, ybt)

    cw = cx_vecs.reshape(NSC, W)
    cx_sc = jnp.min(jnp.minimum(_lo_f32(cw), _hi_f32(cw)), axis=1)
    cham_x_sum = jnp.sum(cx_sc) + cxs_tc[0, 0]

    lo_m = jnp.min(_lo_f32(cy_sc), axis=0)
    hi_m = jnp.min(_hi_f32(cy_sc), axis=0)
    cham_y_sum = (jnp.sum(jnp.minimum(lo_m, cyt[:NP]))
                  + jnp.sum(jnp.minimum(hi_m, cyt[NP:])))
    loss = (cham_x_sum + cham_y_sum) / N
    return (loss, jnp.zeros((1, N), jnp.float32))
